# baseline (speedup 1.0000x reference)
import functools

import jax
import jax.numpy as jnp
from jax import lax
from jax.experimental import pallas as pl
from jax.experimental.pallas import tpu as pltpu
from jax.experimental.pallas import tpu_sc as plsc

EMBED = 64
NC = 2
NS = 16
NW = NC * NS
SB = 40
NBUF = 4

_MESH = dict(core_axis_name="c", subcore_axis_name="s")


@functools.partial(jax.jit, static_argnames=("n_rows",))
def _sc_gather(table, idx_even, idx_odd, n_rows):
    half = n_rows // 2
    s_per_w = half // NW
    n_steps = s_per_w // SB
    n_outer = n_steps // NBUF

    @functools.partial(
        pl.kernel,
        mesh=plsc.VectorSubcoreMesh(**_MESH),
        out_type=jax.ShapeDtypeStruct((half, 128), jnp.float32),
        scratch_types=[
            pltpu.VMEM((s_per_w,), jnp.int32),
            pltpu.VMEM((s_per_w,), jnp.int32),
            *[pltpu.VMEM((SB, EMBED), jnp.float32) for _ in range(2 * NBUF)],
            pltpu.SemaphoreType.DMA,
            pltpu.SemaphoreType.DMA,
        ],
        compiler_params=pltpu.CompilerParams(use_tc_tiling_on_sc=False),
    )
    def k(table_hbm, ie_hbm, io_hbm, out_hbm, ie_v, io_v, *bufs_and_sems):
        bufs = bufs_and_sems[: 2 * NBUF]
        sem_g, sem_w = bufs_and_sems[2 * NBUF:]
        w = lax.axis_index("s") * NC + lax.axis_index("c")
        srow0 = w * s_per_w
        pltpu.sync_copy(ie_hbm.at[pl.ds(srow0, s_per_w)], ie_v)
        pltpu.sync_copy(io_hbm.at[pl.ds(srow0, s_per_w)], io_v)

        def wait_one_write():
            pltpu.make_async_copy(
                bufs[0], out_hbm.at[pl.ds(0, SB), pl.ds(0, EMBED)], sem_w
            ).wait()

        def outer(g, _):
            descs = []
            for b in range(NBUF):
                q = g * NBUF + b

                @pl.when(g >= 1)
                def _():
                    wait_one_write()
                    wait_one_write()

                off = pl.multiple_of(q * SB, SB)
                for u, iv in ((0, ie_v), (1, io_v)):
                    desc = pltpu.make_async_copy(
                        table_hbm.at[iv.at[pl.ds(off, SB)]],
                        bufs[2 * b + u],
                        sem_g,
                    )
                    desc.start()
                    descs.append(desc)
            for b in range(NBUF):
                q = g * NBUF + b
                row0 = pl.multiple_of(srow0 + q * SB, SB)
                for u in (0, 1):
                    descs[2 * b + u].wait()
                    pltpu.make_async_copy(
                        bufs[2 * b + u],
                        out_hbm.at[pl.ds(row0, SB), pl.ds(u * EMBED, EMBED)],
                        sem_w,
                    ).start()
            return 0

        lax.fori_loop(0, n_outer, outer, 0)
        for _ in range(2 * NBUF):
            wait_one_write()

    return k(table, idx_even, idx_odd)


def kernel(locations, table):
    batch, hist = locations.shape
    n_rows = batch * hist
    lflat = locations.reshape(-1)
    staged = _sc_gather(table, lflat[0::2], lflat[1::2], n_rows)
    return staged.reshape(batch, hist, EMBED)


# speedup vs baseline: 1.0815x; 1.0815x over previous
import functools

import jax
import jax.numpy as jnp
from jax import lax
from jax.experimental import pallas as pl
from jax.experimental.pallas import tpu as pltpu
from jax.experimental.pallas import tpu_sc as plsc

EMBED = 64
NC = 2
NS = 16
NW = NC * NS
NB = 8
NBUF = 2


@functools.partial(jax.jit, static_argnames=("batch", "hist"))
def _sc_gather(table, locations, batch, hist):
    b_per_w = batch // NW
    n_steps = b_per_w // NB
    n_outer = n_steps // NBUF
    mesh = plsc.VectorSubcoreMesh(core_axis_name="c", subcore_axis_name="s")

    @functools.partial(
        pl.kernel,
        mesh=mesh,
        out_type=jax.ShapeDtypeStruct((batch, hist, EMBED), jnp.float32),
        scratch_types=[
            pltpu.VMEM((b_per_w, hist), jnp.int32),
            *[pltpu.VMEM((NB, hist, EMBED), jnp.float32) for _ in range(NBUF)],
            pltpu.SemaphoreType.DMA,
            pltpu.SemaphoreType.DMA,
        ],
        compiler_params=pltpu.CompilerParams(use_tc_tiling_on_sc=False),
    )
    def k(table_hbm, idx_hbm, out_hbm, idx_v, *bufs_and_sems):
        bufs = bufs_and_sems[:NBUF]
        sem_g, sem_w = bufs_and_sems[NBUF:]
        wid = lax.axis_index("s") * NC + lax.axis_index("c")
        base = wid * b_per_w
        pltpu.sync_copy(idx_hbm.at[pl.ds(base, b_per_w)], idx_v)

        def wait_one_write():
            pltpu.make_async_copy(
                bufs[0], out_hbm.at[pl.ds(base, NB)], sem_w
            ).wait()

        def outer(g, _):
            descs = []
            for b in range(NBUF):
                t = g * NBUF + b

                @pl.when(g >= 1)
                def _():
                    wait_one_write()

                for j in range(NB):
                    desc = pltpu.make_async_copy(
                        table_hbm.at[idx_v.at[t * NB + j]],
                        bufs[b].at[j],
                        sem_g,
                    )
                    desc.start()
                    descs.append(desc)
            for b in range(NBUF):
                t = g * NBUF + b
                for j in range(NB):
                    descs[b * NB + j].wait()
                pltpu.make_async_copy(
                    bufs[b], out_hbm.at[pl.ds(base + t * NB, NB)], sem_w
                ).start()
            return 0

        lax.fori_loop(0, n_outer, outer, 0)
        for _ in range(NBUF):
            wait_one_write()

    return k(table, locations)


def kernel(locations, table):
    batch, hist = locations.shape
    return _sc_gather(table, locations, batch, hist)
